# baseline (speedup 1.0000x reference)
import functools

import numpy as np
import jax
import jax.numpy as jnp
from jax import lax
from jax.experimental import pallas as pl
from jax.experimental.pallas import tpu as pltpu
from jax.experimental.pallas import tpu_sc as plsc

M = 1_000_000
B = 16_384
FR = 5
CD = 4
ROW = FR * CD
NPF = 32
OUT = ROW * NPF

TPAD = 1_000_064
PLANE = (TPAD // 128) * 512

NC, NS = 2, 16
DUMP = M
INIT_BUF = 7_936
INIT_REPS = 8
INIT_CHUNK = INIT_BUF * INIT_REPS
MPAD = NS * INIT_CHUNK

KPT = B // NS
KCH = KPT // 128
JPT = B // (NC * NS)
JCH = JPT // 128


def _sc_body(rti_hbm, rdi_hbm, ti_hbm, trkf_hbm, detf_hbm,
             trkf_out, detf_out, kval_out,
             aux, initbuf, rti2d, kvals, w2d, newidx,
             ti2d, ti1d, kbuf1d, kc2d, dj1d, fidx_d, fidx_t,
             detfbuf, trkfbuf, sem, sem2):
    cid = lax.axis_index("c")
    sid = lax.axis_index("s")
    wid = cid * NS + sid
    kbase = sid * KPT
    jbase = wid * JPT

    iota = lax.iota(jnp.int32, 16)
    neg1 = jnp.full((16,), -1, jnp.int32)

    def _fill(o, carry):
        for u in range(16):
            initbuf[pl.ds(o * 256 + u * 16, 16)] = neg1
        return carry
    lax.fori_loop(0, INIT_BUF // 256, _fill, 0)
    hs = [pltpu.async_copy(
        initbuf, aux.at[pl.ds(sid * INIT_CHUNK + r * INIT_BUF, INIT_BUF)],
        sem) for r in range(INIT_REPS)]
    hs2 = [pltpu.async_copy(rti_hbm.at[pl.ds(kbase + j * 128, 128)],
                            rti2d.at[j], sem2) for j in range(KCH)]
    hs2 += [pltpu.async_copy(ti_hbm.at[pl.ds(jbase + c * 128, 128)],
                             ti2d.at[c], sem2) for c in range(JCH)]
    hs2.append(pltpu.async_copy(ti_hbm.at[pl.ds(jbase, JPT)],
                                ti1d.at[pl.ds(8, JPT)], sem2))
    for j in range(KCH):
        for i in range(8):
            kvals[pl.ds(j * 128 + i * 16, 16)] = iota + (kbase + j * 128 + i * 16)
    for h in hs:
        h.wait()
    for h in hs2:
        h.wait()
    plsc.subcore_barrier()

    hs = [pltpu.async_copy(kvals.at[pl.ds(j * 128, 128)],
                           aux.at[rti2d.at[j]], sem) for j in range(KCH)]
    for h in hs:
        h.wait()
    plsc.subcore_barrier()

    for _ in range(3):
        hs = [pltpu.async_copy(aux.at[rti2d.at[j]], w2d.at[j], sem)
              for j in range(KCH)]
        for h in hs:
            h.wait()
        for j in range(KCH):
            for i in range(8):
                rv = rti2d[j, pl.ds(i * 16, 16)]
                wv = w2d[j, pl.ds(i * 16, 16)]
                kv = kvals[pl.ds(j * 128 + i * 16, 16)]
                newidx[j, pl.ds(i * 16, 16)] = jnp.where(kv > wv, rv, DUMP)
        plsc.subcore_barrier()
        hs = [pltpu.async_copy(kvals.at[pl.ds(j * 128, 128)],
                               aux.at[newidx.at[j]], sem) for j in range(KCH)]
        for h in hs:
            h.wait()
        plsc.subcore_barrier()

    hs = [pltpu.async_copy(aux.at[ti2d.at[c]],
                           kbuf1d.at[pl.ds(c * 128, 128)], sem)
          for c in range(JCH)]
    for h in hs:
        h.wait()
    for c in range(JCH):
        for i in range(8):
            kv = kbuf1d[pl.ds(c * 128 + i * 16, 16)]
            kc2d[c, pl.ds(i * 16, 16)] = jnp.maximum(kv, 0)
    hs = [pltpu.async_copy(rdi_hbm.at[kc2d.at[c]],
                           dj1d.at[pl.ds(c * 128, 128)], sem)
          for c in range(JCH)]
    for h in hs:
        h.wait()

    for c in range(JCH):
        for s in range(CD):
            for u in range(8):
                e = iota + (s * 128 + u * 16)
                ediv = ((s * 128 + u * 16) >> 2) + (iota >> 2)
                dj = plsc.load_gather(dj1d, [c * 128 + ediv])
                fidx_d[pl.ds((c * CD + s) * 128 + u * 16, 16)] = (
                    dj * CD + (e & (CD - 1)))

    def _tchunk(c, carry):
        def _ts(s, carry2):
            for u in range(8):
                e = iota + (s * 128 + u * 16)
                ediv = (e * 3277) >> 16
                col = e - ediv * ROW
                t = plsc.load_gather(ti1d, [c * 128 + ediv + 8])
                fidx_t[pl.ds((c * ROW + s) * 128 + u * 16, 16)] = (
                    (col >> 2) * PLANE + ((t >> 7) << 9)
                    + ((col & 3) << 7) + (t & 127))
            return carry2
        return lax.fori_loop(0, ROW, _ts, c, unroll=False)

    lax.fori_loop(0, JCH, _tchunk, 0, unroll=False)

    hs = []
    for g in range(JCH * CD):
        hs.append(pltpu.async_copy(
            detf_hbm.at[fidx_d.at[pl.ds(g * 128, 128)]],
            detfbuf.at[pl.ds(g * 128, 128)], sem))
    for g in range(JCH * ROW):
        hs.append(pltpu.async_copy(
            trkf_hbm.at[fidx_t.at[pl.ds(g * 128, 128)]],
            trkfbuf.at[pl.ds(g * 128, 128)], sem2))
    for h in hs:
        h.wait()

    hs = [pltpu.async_copy(kbuf1d, kval_out.at[pl.ds(jbase, JPT)], sem),
          pltpu.async_copy(detfbuf, detf_out.at[pl.ds(jbase * CD, JPT * CD)],
                           sem),
          pltpu.async_copy(trkfbuf, trkf_out.at[pl.ds(jbase * ROW, JPT * ROW)],
                           sem)]
    for h in hs:
        h.wait()


_sc_gather = functools.partial(
    pl.kernel,
    mesh=plsc.VectorSubcoreMesh(core_axis_name="c", subcore_axis_name="s"),
    out_type=[
        jax.ShapeDtypeStruct((B * ROW,), jnp.float32),
        jax.ShapeDtypeStruct((B * CD,), jnp.float32),
        jax.ShapeDtypeStruct((B,), jnp.int32),
    ],
    scratch_types=[
        pltpu.VMEM_SHARED((MPAD,), jnp.int32),
        pltpu.VMEM((INIT_BUF,), jnp.int32),
        pltpu.VMEM((KCH, 128), jnp.int32),
        pltpu.VMEM((KPT,), jnp.int32),
        pltpu.VMEM((KCH, 128), jnp.int32),
        pltpu.VMEM((KCH, 128), jnp.int32),
        pltpu.VMEM((JCH, 128), jnp.int32),
        pltpu.VMEM((JPT + 8,), jnp.int32),
        pltpu.VMEM((JPT,), jnp.int32),
        pltpu.VMEM((JCH, 128), jnp.int32),
        pltpu.VMEM((JPT,), jnp.int32),
        pltpu.VMEM((JPT * CD,), jnp.int32),
        pltpu.VMEM((JPT * ROW,), jnp.int32),
        pltpu.VMEM((JPT * CD,), jnp.float32),
        pltpu.VMEM((JPT * ROW,), jnp.float32),
        pltpu.SemaphoreType.DMA,
        pltpu.SemaphoreType.DMA,
    ],
    compiler_params=pltpu.CompilerParams(
        use_tc_tiling_on_sc=False, needs_layout_passes=False),
)(_sc_body)


BLK = 256


def _enc_body(trk_ref, det_ref, k_ref, out_ref):
    trk = trk_ref[...]
    det = det_ref[...]
    kv = k_ref[...]
    det20 = jnp.reshape(
        jnp.broadcast_to(det[:, None, :], (BLK, FR, CD)), (BLK, ROW))
    x = jnp.where(kv >= 0, det20, trk)
    xr = jnp.reshape(
        jnp.broadcast_to(x[:, :, None], (BLK, ROW, NPF)), (BLK, OUT))
    q = lax.broadcasted_iota(jnp.int32, (1, OUT), 1)
    e = (q % 16).astype(jnp.float32) * jnp.float32(2.0 / 32.0)
    dim_t = jnp.exp(e * jnp.float32(np.log(10000.0)))
    phase = jnp.where((q % 32) < 16, jnp.float32(0.0),
                      jnp.float32(np.pi / 2))
    out_ref[...] = jnp.cos(xr * (jnp.float32(2.0 * np.pi) / dim_t) - phase)


_encode = pl.pallas_call(
    _enc_body,
    grid=(B // BLK,),
    in_specs=[
        pl.BlockSpec((BLK, ROW), lambda g: (g, 0)),
        pl.BlockSpec((BLK, CD), lambda g: (g, 0)),
        pl.BlockSpec((BLK, 1), lambda g: (g, 0)),
    ],
    out_specs=pl.BlockSpec((BLK, OUT), lambda g: (g, 0)),
    out_shape=jax.ShapeDtypeStruct((B, OUT), jnp.float32),
)


def kernel(tracklets, detections, replace_track_indices, replace_det_indices,
           tracklets_indices):
    tpad = jnp.pad(tracklets, ((0, TPAD - M), (0, 0), (0, 0)))
    table = (tpad.reshape(TPAD // 128, 128, FR, CD)
             .transpose(2, 0, 3, 1).reshape(FR * PLANE))
    trkf, detf, kval = _sc_gather(
        replace_track_indices, replace_det_indices, tracklets_indices,
        table, detections.reshape(B * CD))
    return _encode(trkf.reshape(B, ROW), detf.reshape(B, CD),
                   kval.reshape(B, 1))


# speedup vs baseline: 1.0993x; 1.0993x over previous
import functools

import numpy as np
import jax
import jax.numpy as jnp
from jax import lax
from jax.experimental import pallas as pl
from jax.experimental.pallas import tpu as pltpu
from jax.experimental.pallas import tpu_sc as plsc

M = 1_000_000
B = 16_384
FR = 5
CD = 4
ROW = FR * CD
NPF = 32
OUT = ROW * NPF

TPAD = 1_000_064
PLANE = (TPAD // 128) * 512

NC, NS = 2, 16
DUMP = M
INIT_BUF = 7_936
INIT_REPS = 8
INIT_CHUNK = INIT_BUF * INIT_REPS
MPAD = NS * INIT_CHUNK

KPT = B // NS
KCH = KPT // 128
JPT = B // (NC * NS)
JCH = JPT // 128


def _sc_body(rti_hbm, rdi_hbm, ti_hbm, trkf_hbm, detf_hbm,
             trkf_out, detf_out, kval_out,
             aux, initbuf, rti2d, kvals, w_buf, newidx,
             ti2d, ti1d, kbuf, kc1d, djbuf, fidx_d, fidx_t,
             detfbuf, trkfbuf):
    cid = lax.axis_index("c")
    sid = lax.axis_index("s")
    wid = cid * NS + sid

    iota = lax.iota(jnp.int32, 16)
    neg1 = jnp.full((16,), -1, jnp.int32)

    def _fill(o, carry):
        for u in range(16):
            initbuf[pl.ds(o * 256 + u * 16, 16)] = neg1
        return carry
    lax.fori_loop(0, INIT_BUF // 256, _fill, 0)
    for r in range(INIT_REPS):
        pltpu.sync_copy(
            initbuf, aux.at[pl.ds(sid * INIT_CHUNK + r * INIT_BUF, INIT_BUF)])

    kbase = sid * KPT
    for j in range(KCH):
        pltpu.sync_copy(rti_hbm.at[pl.ds(kbase + j * 128, 128)], rti2d.at[j])
    for j in range(KCH):
        for i in range(8):
            kvals[pl.ds(j * 128 + i * 16, 16)] = iota + (kbase + j * 128 + i * 16)
    plsc.subcore_barrier()

    for j in range(KCH):
        pltpu.sync_copy(kvals.at[pl.ds(j * 128, 128)], aux.at[rti2d.at[j]])
    plsc.subcore_barrier()

    for _ in range(3):
        for j in range(KCH):
            pltpu.sync_copy(aux.at[rti2d.at[j]], w_buf)
            for i in range(8):
                rv = rti2d[j, pl.ds(i * 16, 16)]
                wv = w_buf[pl.ds(i * 16, 16)]
                kv = kvals[pl.ds(j * 128 + i * 16, 16)]
                newidx[j, pl.ds(i * 16, 16)] = jnp.where(kv > wv, rv, DUMP)
        plsc.subcore_barrier()
        for j in range(KCH):
            pltpu.sync_copy(kvals.at[pl.ds(j * 128, 128)], aux.at[newidx.at[j]])
        plsc.subcore_barrier()

    jbase = wid * JPT
    for c in range(JCH):
        pltpu.sync_copy(ti_hbm.at[pl.ds(jbase + c * 128, 128)], ti2d.at[c])
    pltpu.sync_copy(ti_hbm.at[pl.ds(jbase, JPT)], ti1d.at[pl.ds(8, JPT)])
    for c in range(JCH):
        pltpu.sync_copy(aux.at[ti2d.at[c]], kbuf)
        for i in range(8):
            kv = kbuf[pl.ds(i * 16, 16)]
            kc1d[pl.ds(i * 16, 16)] = jnp.maximum(kv, 0)
        pltpu.sync_copy(rdi_hbm.at[kc1d], djbuf)
        for s in range(CD):
            for u in range(8):
                e = iota + (s * 128 + u * 16)
                ediv = ((s * 128 + u * 16) >> 2) + (iota >> 2)
                dj = plsc.load_gather(djbuf, [ediv])
                fidx_d[s, pl.ds(u * 16, 16)] = dj * CD + (e & (CD - 1))
            pltpu.sync_copy(detf_hbm.at[fidx_d.at[s]],
                            detfbuf.at[pl.ds(s * 128, 128)])
        for s in range(ROW):
            for u in range(8):
                e = iota + (s * 128 + u * 16)
                ediv = (e * 3277) >> 16
                col = e - ediv * ROW
                t = plsc.load_gather(ti1d, [c * 128 + ediv + 8])
                fidx_t[s, pl.ds(u * 16, 16)] = (
                    (col >> 2) * PLANE + ((t >> 7) << 9)
                    + ((col & 3) << 7) + (t & 127))
            pltpu.sync_copy(trkf_hbm.at[fidx_t.at[s]],
                            trkfbuf.at[pl.ds(s * 128, 128)])
        pltpu.sync_copy(kbuf, kval_out.at[pl.ds(jbase + c * 128, 128)])
        pltpu.sync_copy(detfbuf,
                        detf_out.at[pl.ds((jbase + c * 128) * CD, 128 * CD)])
        pltpu.sync_copy(trkfbuf,
                        trkf_out.at[pl.ds((jbase + c * 128) * ROW, 128 * ROW)])


_sc_gather = functools.partial(
    pl.kernel,
    mesh=plsc.VectorSubcoreMesh(core_axis_name="c", subcore_axis_name="s"),
    out_type=[
        jax.ShapeDtypeStruct((B * ROW,), jnp.float32),
        jax.ShapeDtypeStruct((B * CD,), jnp.float32),
        jax.ShapeDtypeStruct((B,), jnp.int32),
    ],
    scratch_types=[
        pltpu.VMEM_SHARED((MPAD,), jnp.int32),
        pltpu.VMEM((INIT_BUF,), jnp.int32),
        pltpu.VMEM((KCH, 128), jnp.int32),
        pltpu.VMEM((KPT,), jnp.int32),
        pltpu.VMEM((128,), jnp.int32),
        pltpu.VMEM((KCH, 128), jnp.int32),
        pltpu.VMEM((JCH, 128), jnp.int32),
        pltpu.VMEM((JPT + 8,), jnp.int32),
        pltpu.VMEM((128,), jnp.int32),
        pltpu.VMEM((128,), jnp.int32),
        pltpu.VMEM((128,), jnp.int32),
        pltpu.VMEM((CD, 128), jnp.int32),
        pltpu.VMEM((ROW, 128), jnp.int32),
        pltpu.VMEM((128 * CD,), jnp.float32),
        pltpu.VMEM((128 * ROW,), jnp.float32),
    ],
    compiler_params=pltpu.CompilerParams(
        use_tc_tiling_on_sc=False, needs_layout_passes=False),
)(_sc_body)


BLK = 256

_EXPAND = np.zeros((ROW, OUT), np.float32)
_EXPAND[np.arange(OUT) // NPF, np.arange(OUT)] = 1.0


def _enc_body(trk_ref, det_ref, k_ref, exp_ref, out_ref):
    trk = trk_ref[...]
    det = det_ref[...]
    kv = k_ref[...]
    det20 = jnp.reshape(
        jnp.broadcast_to(det[:, None, :], (BLK, FR, CD)), (BLK, ROW))
    x = jnp.where(kv >= 0, det20, trk)
    xr = jnp.dot(x, exp_ref[...],
                 precision=lax.Precision.HIGHEST)
    q = lax.broadcasted_iota(jnp.int32, (1, OUT), 1)
    e = (q % 16).astype(jnp.float32) * jnp.float32(2.0 / 32.0)
    dim_t = jnp.exp(e * jnp.float32(np.log(10000.0)))
    phase = jnp.where((q % 32) < 16, jnp.float32(0.0),
                      jnp.float32(np.pi / 2))
    out_ref[...] = jnp.cos(xr * (jnp.float32(2.0 * np.pi) / dim_t) - phase)


_encode = pl.pallas_call(
    _enc_body,
    grid=(B // BLK,),
    in_specs=[
        pl.BlockSpec((BLK, ROW), lambda g: (g, 0)),
        pl.BlockSpec((BLK, CD), lambda g: (g, 0)),
        pl.BlockSpec((BLK, 1), lambda g: (g, 0)),
        pl.BlockSpec((ROW, OUT), lambda g: (0, 0)),
    ],
    out_specs=pl.BlockSpec((BLK, OUT), lambda g: (g, 0)),
    out_shape=jax.ShapeDtypeStruct((B, OUT), jnp.float32),
)


def kernel(tracklets, detections, replace_track_indices, replace_det_indices,
           tracklets_indices):
    tpad = jnp.pad(tracklets, ((0, TPAD - M), (0, 0), (0, 0)))
    table = (tpad.reshape(TPAD // 128, 128, FR, CD)
             .transpose(2, 0, 3, 1).reshape(FR * PLANE))
    trkf, detf, kval = _sc_gather(
        replace_track_indices, replace_det_indices, tracklets_indices,
        table, detections.reshape(B * CD))
    return _encode(trkf.reshape(B, ROW), detf.reshape(B, CD),
                   kval.reshape(B, 1), jnp.asarray(_EXPAND))


# speedup vs baseline: 1.1018x; 1.0022x over previous
import functools

import numpy as np
import jax
import jax.numpy as jnp
from jax import lax
from jax.experimental import pallas as pl
from jax.experimental.pallas import tpu as pltpu
from jax.experimental.pallas import tpu_sc as plsc

M = 1_000_000
B = 16_384
FR = 5
CD = 4
ROW = FR * CD
NPF = 32
OUT = ROW * NPF

TPAD = 1_000_064
PLANE = (TPAD // 128) * 512

NC, NS = 2, 16
DUMP = M
INIT_BUF = 7_936
INIT_REPS = 8
INIT_CHUNK = INIT_BUF * INIT_REPS
MPAD = NS * INIT_CHUNK

KPT = B // NS
KCH = KPT // 128
JPT = B // (NC * NS)
JCH = JPT // 128


def _sc_body(rti_hbm, rdi_hbm, ti_hbm, trkf_hbm, detf_hbm,
             trkf_out, detf_out, kval_out,
             aux, initbuf, rti2d, kvals, w_buf, newidx,
             ti2d, ti1d, kbuf, kc1d, djbuf, fidx_d, fidx_t,
             detfbuf, trkfbuf):
    cid = lax.axis_index("c")
    sid = lax.axis_index("s")
    wid = cid * NS + sid

    iota = lax.iota(jnp.int32, 16)
    neg1 = jnp.full((16,), -1, jnp.int32)

    def _fill(o, carry):
        for u in range(16):
            initbuf[pl.ds(o * 256 + u * 16, 16)] = neg1
        return carry
    lax.fori_loop(0, INIT_BUF // 256, _fill, 0)
    for r in range(INIT_REPS):
        pltpu.sync_copy(
            initbuf, aux.at[pl.ds(sid * INIT_CHUNK + r * INIT_BUF, INIT_BUF)])

    kbase = sid * KPT
    for j in range(KCH):
        pltpu.sync_copy(rti_hbm.at[pl.ds(kbase + j * 128, 128)], rti2d.at[j])
    for j in range(KCH):
        for i in range(8):
            kvals[pl.ds(j * 128 + i * 16, 16)] = iota + (kbase + j * 128 + i * 16)
    plsc.subcore_barrier()

    for j in range(KCH):
        pltpu.sync_copy(kvals.at[pl.ds(j * 128, 128)], aux.at[rti2d.at[j]])
    plsc.subcore_barrier()

    for _ in range(3):
        for j in range(KCH):
            pltpu.sync_copy(aux.at[rti2d.at[j]], w_buf)
            for i in range(8):
                rv = rti2d[j, pl.ds(i * 16, 16)]
                wv = w_buf[pl.ds(i * 16, 16)]
                kv = kvals[pl.ds(j * 128 + i * 16, 16)]
                newidx[j, pl.ds(i * 16, 16)] = jnp.where(kv > wv, rv, DUMP)
        plsc.subcore_barrier()
        for j in range(KCH):
            pltpu.sync_copy(kvals.at[pl.ds(j * 128, 128)], aux.at[newidx.at[j]])
        plsc.subcore_barrier()

    jbase = wid * JPT
    for c in range(JCH):
        pltpu.sync_copy(ti_hbm.at[pl.ds(jbase + c * 128, 128)], ti2d.at[c])
    pltpu.sync_copy(ti_hbm.at[pl.ds(jbase, JPT)], ti1d.at[pl.ds(8, JPT)])
    for c in range(JCH):
        pltpu.sync_copy(aux.at[ti2d.at[c]], kbuf)
        for i in range(8):
            kv = kbuf[pl.ds(i * 16, 16)]
            kc1d[pl.ds(i * 16, 16)] = jnp.maximum(kv, 0)
        pltpu.sync_copy(rdi_hbm.at[kc1d], djbuf)
        for s in range(CD):
            for u in range(8):
                e = iota + (s * 128 + u * 16)
                ediv = ((s * 128 + u * 16) >> 2) + (iota >> 2)
                dj = plsc.load_gather(djbuf, [ediv])
                fidx_d[s, pl.ds(u * 16, 16)] = dj * CD + (e & (CD - 1))
            pltpu.sync_copy(detf_hbm.at[fidx_d.at[s]],
                            detfbuf.at[pl.ds(s * 128, 128)])
        for s in range(ROW):
            for u in range(8):
                e = iota + (s * 128 + u * 16)
                ediv = (e * 3277) >> 16
                col = e - ediv * ROW
                t = plsc.load_gather(ti1d, [c * 128 + ediv + 8])
                fidx_t[s, pl.ds(u * 16, 16)] = (
                    (col >> 2) * PLANE + ((t >> 7) << 9)
                    + ((col & 3) << 7) + (t & 127))
            pltpu.sync_copy(trkf_hbm.at[fidx_t.at[s]],
                            trkfbuf.at[pl.ds(s * 128, 128)])
        pltpu.sync_copy(kbuf, kval_out.at[pl.ds(jbase + c * 128, 128)])
        pltpu.sync_copy(detfbuf,
                        detf_out.at[pl.ds((jbase + c * 128) * CD, 128 * CD)])
        pltpu.sync_copy(trkfbuf,
                        trkf_out.at[pl.ds((jbase + c * 128) * ROW, 128 * ROW)])


_sc_gather = functools.partial(
    pl.kernel,
    mesh=plsc.VectorSubcoreMesh(core_axis_name="c", subcore_axis_name="s"),
    out_type=[
        jax.ShapeDtypeStruct((B * ROW,), jnp.float32),
        jax.ShapeDtypeStruct((B * CD,), jnp.float32),
        jax.ShapeDtypeStruct((B,), jnp.int32),
    ],
    scratch_types=[
        pltpu.VMEM_SHARED((MPAD,), jnp.int32),
        pltpu.VMEM((INIT_BUF,), jnp.int32),
        pltpu.VMEM((KCH, 128), jnp.int32),
        pltpu.VMEM((KPT,), jnp.int32),
        pltpu.VMEM((128,), jnp.int32),
        pltpu.VMEM((KCH, 128), jnp.int32),
        pltpu.VMEM((JCH, 128), jnp.int32),
        pltpu.VMEM((JPT + 8,), jnp.int32),
        pltpu.VMEM((128,), jnp.int32),
        pltpu.VMEM((128,), jnp.int32),
        pltpu.VMEM((128,), jnp.int32),
        pltpu.VMEM((CD, 128), jnp.int32),
        pltpu.VMEM((ROW, 128), jnp.int32),
        pltpu.VMEM((128 * CD,), jnp.float32),
        pltpu.VMEM((128 * ROW,), jnp.float32),
    ],
    compiler_params=pltpu.CompilerParams(
        use_tc_tiling_on_sc=False, needs_layout_passes=False),
)(_sc_body)


BLK = 512

_EXPAND = np.zeros((ROW, OUT), np.float32)
_EXPAND[np.arange(OUT) // NPF, np.arange(OUT)] = 1.0


def _enc_body(trk_ref, det_ref, k_ref, exp_ref, out_ref):
    trk = trk_ref[...]
    det = det_ref[...]
    kv = k_ref[...]
    det20 = jnp.reshape(
        jnp.broadcast_to(det[:, None, :], (BLK, FR, CD)), (BLK, ROW))
    x = jnp.where(kv >= 0, det20, trk)
    xr = jnp.dot(x, exp_ref[...],
                 precision=lax.Precision.HIGHEST)
    q = lax.broadcasted_iota(jnp.int32, (1, OUT), 1)
    e = (q % 16).astype(jnp.float32) * jnp.float32(2.0 / 32.0)
    dim_t = jnp.exp(e * jnp.float32(np.log(10000.0)))
    phase = jnp.where((q % 32) < 16, jnp.float32(0.0),
                      jnp.float32(np.pi / 2))
    out_ref[...] = jnp.cos(xr * (jnp.float32(2.0 * np.pi) / dim_t) - phase)


_encode = pl.pallas_call(
    _enc_body,
    grid=(B // BLK,),
    in_specs=[
        pl.BlockSpec((BLK, ROW), lambda g: (g, 0)),
        pl.BlockSpec((BLK, CD), lambda g: (g, 0)),
        pl.BlockSpec((BLK, 1), lambda g: (g, 0)),
        pl.BlockSpec((ROW, OUT), lambda g: (0, 0)),
    ],
    out_specs=pl.BlockSpec((BLK, OUT), lambda g: (g, 0)),
    out_shape=jax.ShapeDtypeStruct((B, OUT), jnp.float32),
)


def kernel(tracklets, detections, replace_track_indices, replace_det_indices,
           tracklets_indices):
    tpad = jnp.pad(tracklets, ((0, TPAD - M), (0, 0), (0, 0)))
    table = (tpad.reshape(TPAD // 128, 128, FR, CD)
             .transpose(2, 0, 3, 1).reshape(FR * PLANE))
    trkf, detf, kval = _sc_gather(
        replace_track_indices, replace_det_indices, tracklets_indices,
        table, detections.reshape(B * CD))
    return _encode(trkf.reshape(B, ROW), detf.reshape(B, CD),
                   kval.reshape(B, 1), jnp.asarray(_EXPAND))
